# baseline (speedup 1.0000x reference)
import functools

import jax
import jax.numpy as jnp
from jax import lax
from jax.experimental import pallas as pl
from jax.experimental.pallas import tpu as pltpu
from jax.experimental.pallas import tpu_sc as plsc

_LANES = 16
_NUM_CORES = 1


@functools.lru_cache(maxsize=None)
def _build(n, num_types):
    info = plsc.get_sparse_core_info()
    nc, ns = _NUM_CORES, info.num_subcores
    nw = nc * ns
    chunk = -(-n // (_LANES * nw)) * _LANES
    n_vec = chunk // _LANES
    assert n % 8 == 0 and n >= chunk

    mesh = plsc.VectorSubcoreMesh(
        core_axis_name="c", subcore_axis_name="s", num_cores=nc)

    @functools.partial(
        pl.kernel,
        mesh=mesh,
        out_type=jax.ShapeDtypeStruct((n,), jnp.float32),
        compiler_params=pltpu.CompilerParams(
            needs_layout_passes=False, skip_device_barrier=True),
        scratch_types=[
            pltpu.VMEM((chunk,), jnp.int32),
            pltpu.VMEM((chunk,), jnp.float32),
            pltpu.VMEM((chunk,), jnp.float32),
            pltpu.VMEM((num_types,), jnp.float32),
            pltpu.VMEM((num_types,), jnp.float32),
            pltpu.SemaphoreType.DMA,
        ],
    )
    def scale_shift(energy_hbm, types_hbm, scales_hbm, shifts_hbm, out_hbm,
                    idx_v, e_v, o_v, sc_v, sh_v, sem):
        wid = lax.axis_index("s") * nc + lax.axis_index("c")
        base = jnp.minimum(wid * chunk, n - chunk)
        cps = (
            pltpu.make_async_copy(scales_hbm, sc_v, sem),
            pltpu.make_async_copy(shifts_hbm, sh_v, sem),
            pltpu.make_async_copy(
                types_hbm.at[pl.ds(base, chunk)], idx_v, sem),
            pltpu.make_async_copy(
                energy_hbm.at[pl.ds(base, chunk)], e_v, sem),
        )
        for cp in cps:
            cp.start()
        for cp in cps:
            cp.wait()

        @plsc.parallel_loop(0, n_vec, unroll=4)
        def _vec(j):
            off = j * _LANES
            idx16 = idx_v[pl.ds(off, _LANES)]
            e16 = e_v[pl.ds(off, _LANES)]
            sc16 = plsc.load_gather(sc_v, [idx16])
            sh16 = plsc.load_gather(sh_v, [idx16])
            o_v[pl.ds(off, _LANES)] = sh16 + sc16 * e16

        pltpu.sync_copy(o_v, out_hbm.at[pl.ds(base, chunk)])

    return scale_shift


def kernel(atomic_energy, atom_types, scales, shifts):
    n = atom_types.shape[0]
    num_types = scales.shape[0]
    energy = atomic_energy.reshape(n).astype(jnp.float32)
    types = atom_types.astype(jnp.int32)
    out = _build(n, num_types)(energy, types, scales, shifts)
    return out.reshape(n, 1)
